# Initial kernel scaffold; baseline (speedup 1.0000x reference)
import functools

import jax
import jax.numpy as jnp
from jax import lax
from jax.experimental import pallas as pl
from jax.experimental.pallas import tpu as pltpu
from jax.experimental.pallas import tpu_sc as plsc

N = 10000
D = 128
NC = 2
NS = 16
NW = NC * NS
CHUNK = 128
N_ACC = 10240
ROWS_PER_TILE = N_ACC // NS
ZCH = ROWS_PER_TILE // CHUNK


def _spmm_sc(support, src_r, dst_r):
    C = src_r.shape[1]
    mesh = plsc.VectorSubcoreMesh(core_axis_name="c", subcore_axis_name="s")

    @functools.partial(
        pl.kernel,
        out_type=jax.ShapeDtypeStruct((NC, N_ACC, D), jnp.float32),
        mesh=mesh,
        scratch_types=[
            pltpu.VMEM((C, CHUNK), jnp.int32),
            pltpu.VMEM((C, CHUNK), jnp.int32),
            pltpu.VMEM((CHUNK, D), jnp.float32),
            pltpu.VMEM((CHUNK, D), jnp.float32),
            pltpu.VMEM_SHARED((N_ACC, D), jnp.float32),
            pltpu.SemaphoreType.DMA,
        ],
    )
    def spmm(support_hbm, src_hbm, dst_hbm, out_hbm, src_v, dst_v, buf, zbuf,
             acc, sem):
        c = lax.axis_index("c")
        s = lax.axis_index("s")
        wid = s * NC + c

        def zrow(r, _):
            def zcol(q, _):
                zbuf[r, pl.ds(q * 16, 16)] = jnp.zeros((16,), jnp.float32)
                return 0
            return lax.fori_loop(0, D // 16, zcol, 0)
        lax.fori_loop(0, CHUNK, zrow, 0)

        def zacc(i, _):
            off = s * ROWS_PER_TILE + i * CHUNK
            pltpu.sync_copy(zbuf, acc.at[pl.ds(off, CHUNK)])
            return 0
        lax.fori_loop(0, ZCH, zacc, 0)
        plsc.subcore_barrier()

        pltpu.sync_copy(src_hbm.at[wid], src_v)
        pltpu.sync_copy(dst_hbm.at[wid], dst_v)

        def body(j, _):
            pltpu.async_copy(support_hbm.at[src_v.at[j]], buf, sem).wait()
            pltpu.sync_copy(buf, acc.at[dst_v.at[j]], add=True)
            return 0
        lax.fori_loop(0, C, body, 0)
        plsc.subcore_barrier()

        def cout(i, _):
            off = s * ROWS_PER_TILE + i * CHUNK
            pltpu.sync_copy(acc.at[pl.ds(off, CHUNK)], buf)
            pltpu.sync_copy(buf, out_hbm.at[c, pl.ds(off, CHUNK)])
            return 0
        lax.fori_loop(0, ZCH, cout, 0)

    return spmm(support, src_r, dst_r)


def _mm_tc(x, w):
    def body(x_ref, w_ref, o_ref):
        o_ref[...] = jnp.dot(x_ref[...], w_ref[...],
                             preferred_element_type=jnp.float32)
    return pl.pallas_call(
        body,
        out_shape=jax.ShapeDtypeStruct((x.shape[0], w.shape[1]), jnp.float32),
    )(x, w)


def _combine_bn_elu_mm(p, b, x, gamma, beta, w2):
    def body(p_ref, b_ref, x_ref, g_ref, be_ref, w2_ref, h1_ref, s2_ref):
        h = p_ref[0, :N, :] + p_ref[1, :N, :] + x_ref[...] + b_ref[...]
        mean = jnp.mean(h, axis=0, keepdims=True)
        var = jnp.mean((h - mean) * (h - mean), axis=0, keepdims=True)
        hn = g_ref[...] * (h - mean) * lax.rsqrt(var + 1e-5) + be_ref[...]
        h1 = jnp.where(hn > 0, hn, jnp.exp(jnp.minimum(hn, 0.0)) - 1.0)
        h1_ref[...] = h1
        s2_ref[...] = jnp.dot(h1, w2_ref[...],
                              preferred_element_type=jnp.float32)
    return pl.pallas_call(
        body,
        out_shape=(
            jax.ShapeDtypeStruct((N, D), jnp.float32),
            jax.ShapeDtypeStruct((N, D), jnp.float32),
        ),
    )(p, b, x, gamma, beta, w2)


def _combine_final(q, b, h1):
    def body(q_ref, b_ref, h1_ref, o_ref):
        o_ref[...] = q_ref[0, :N, :] + q_ref[1, :N, :] + b_ref[...] + h1_ref[...]
    return pl.pallas_call(
        body,
        out_shape=jax.ShapeDtypeStruct((N, D), jnp.float32),
    )(q, b, h1)


def kernel(features, edge_index, W1, b1, W2, b2, gamma0, beta0):
    E = edge_index.shape[1]
    C = -(-E // (NW * CHUNK))
    E_pad = NW * C * CHUNK
    src = edge_index[0]
    dst = edge_index[1]
    pad = E_pad - E
    src_p = jnp.concatenate([src, jnp.zeros((pad,), jnp.int32)])
    dst_p = jnp.concatenate([dst, jnp.full((pad,), N, jnp.int32)])
    src_r = src_p.reshape(NW, C, CHUNK)
    dst_r = dst_p.reshape(NW, C, CHUNK)

    b1r = b1.reshape(1, D)
    b2r = b2.reshape(1, D)
    g0 = gamma0.reshape(1, D)
    be0 = beta0.reshape(1, D)

    s1 = _mm_tc(features, W1)
    p = _spmm_sc(s1, src_r, dst_r)
    h1, s2 = _combine_bn_elu_mm(p, b1r, features, g0, be0, W2)
    q = _spmm_sc(s2, src_r, dst_r)
    h2 = _combine_final(q, b2r, h1)
    return (h1, h2)


# speedup vs baseline: 3.2408x; 3.2408x over previous
import functools

import jax
import jax.numpy as jnp
from jax import lax
from jax.experimental import pallas as pl
from jax.experimental.pallas import tpu as pltpu
from jax.experimental.pallas import tpu_sc as plsc

N = 10000
D = 128
NC = 2
NS = 16
NW = NC * NS
CHUNK = 128
IB = 8
N_ACC = 10240
ROWS_PER_TILE = N_ACC // NS
ZCH = ROWS_PER_TILE // CHUNK


def _spmm_sc(support, src_r, dst_r):
    C = src_r.shape[1]
    G = C // IB
    mesh = plsc.VectorSubcoreMesh(core_axis_name="c", subcore_axis_name="s")

    @functools.partial(
        pl.kernel,
        out_type=jax.ShapeDtypeStruct((NC, N_ACC, D), jnp.float32),
        mesh=mesh,
        scratch_types=[
            pltpu.VMEM((IB, CHUNK), jnp.int32),
            pltpu.VMEM((IB, CHUNK), jnp.int32),
            pltpu.VMEM((CHUNK, D), jnp.float32),
            pltpu.VMEM((CHUNK, D), jnp.float32),
            pltpu.VMEM_SHARED((N_ACC, D), jnp.float32),
            pltpu.SemaphoreType.DMA,
            pltpu.SemaphoreType.DMA,
        ],
    )
    def spmm(support_hbm, src_hbm, dst_hbm, out_hbm, sidx, didx, buf0, buf1,
             acc, sem0, sem1):
        c = lax.axis_index("c")
        s = lax.axis_index("s")
        wid = s * NC + c
        bufs = (buf0, buf1)
        sems = (sem0, sem1)

        def zrow(r, _):
            def zcol(q, _):
                buf0[r, pl.ds(q * 16, 16)] = jnp.zeros((16,), jnp.float32)
                return 0
            return lax.fori_loop(0, D // 16, zcol, 0)
        lax.fori_loop(0, CHUNK, zrow, 0)

        def zacc(i, _):
            off = s * ROWS_PER_TILE + i * CHUNK
            pltpu.sync_copy(buf0, acc.at[pl.ds(off, CHUNK)])
            return 0
        lax.fori_loop(0, ZCH, zacc, 0)
        plsc.subcore_barrier()

        def sup(g, _):
            pltpu.sync_copy(src_hbm.at[wid, pl.ds(g * IB, IB)], sidx)
            pltpu.sync_copy(dst_hbm.at[wid, pl.ds(g * IB, IB)], didx)
            descs = [
                pltpu.async_copy(support_hbm.at[sidx.at[0]], buf0, sem0),
                pltpu.async_copy(support_hbm.at[sidx.at[1]], buf1, sem1),
            ]
            for b in range(IB):
                k = b % 2
                descs[k].wait()
                pltpu.sync_copy(bufs[k], acc.at[didx.at[b]], add=True)
                if b + 2 < IB:
                    descs[k] = pltpu.async_copy(
                        support_hbm.at[sidx.at[b + 2]], bufs[k], sems[k])
            return 0
        lax.fori_loop(0, G, sup, 0)
        plsc.subcore_barrier()

        def cout(i, _):
            off = s * ROWS_PER_TILE + i * CHUNK
            pltpu.sync_copy(acc.at[pl.ds(off, CHUNK)], buf0)
            pltpu.sync_copy(buf0, out_hbm.at[c, pl.ds(off, CHUNK)])
            return 0
        lax.fori_loop(0, ZCH, cout, 0)

    return spmm(support, src_r, dst_r)


def _mm_tc(x, w):
    def body(x_ref, w_ref, o_ref):
        o_ref[...] = jnp.dot(x_ref[...], w_ref[...],
                             preferred_element_type=jnp.float32)
    return pl.pallas_call(
        body,
        out_shape=jax.ShapeDtypeStruct((x.shape[0], w.shape[1]), jnp.float32),
    )(x, w)


def _combine_bn_elu_mm(p, b, x, gamma, beta, w2):
    def body(p_ref, b_ref, x_ref, g_ref, be_ref, w2_ref, h1_ref, s2_ref):
        h = p_ref[0, :N, :] + p_ref[1, :N, :] + x_ref[...] + b_ref[...]
        mean = jnp.mean(h, axis=0, keepdims=True)
        var = jnp.mean((h - mean) * (h - mean), axis=0, keepdims=True)
        hn = g_ref[...] * (h - mean) * lax.rsqrt(var + 1e-5) + be_ref[...]
        h1 = jnp.where(hn > 0, hn, jnp.exp(jnp.minimum(hn, 0.0)) - 1.0)
        h1_ref[...] = h1
        s2_ref[...] = jnp.dot(h1, w2_ref[...],
                              preferred_element_type=jnp.float32)
    return pl.pallas_call(
        body,
        out_shape=(
            jax.ShapeDtypeStruct((N, D), jnp.float32),
            jax.ShapeDtypeStruct((N, D), jnp.float32),
        ),
    )(p, b, x, gamma, beta, w2)


def _combine_final(q, b, h1):
    def body(q_ref, b_ref, h1_ref, o_ref):
        o_ref[...] = q_ref[0, :N, :] + q_ref[1, :N, :] + b_ref[...] + h1_ref[...]
    return pl.pallas_call(
        body,
        out_shape=jax.ShapeDtypeStruct((N, D), jnp.float32),
    )(q, b, h1)


def kernel(features, edge_index, W1, b1, W2, b2, gamma0, beta0):
    E = edge_index.shape[1]
    C = -(-E // (NW * CHUNK * IB)) * IB
    E_pad = NW * C * CHUNK
    src = edge_index[0]
    dst = edge_index[1]
    pad = E_pad - E
    src_p = jnp.concatenate([src, jnp.zeros((pad,), jnp.int32)])
    dst_p = jnp.concatenate([dst, jnp.full((pad,), N, jnp.int32)])
    src_r = src_p.reshape(NW, C, CHUNK)
    dst_r = dst_p.reshape(NW, C, CHUNK)

    b1r = b1.reshape(1, D)
    b2r = b2.reshape(1, D)
    g0 = gamma0.reshape(1, D)
    be0 = beta0.reshape(1, D)

    s1 = _mm_tc(features, W1)
    p = _spmm_sc(s1, src_r, dst_r)
    h1, s2 = _combine_bn_elu_mm(p, b1r, features, g0, be0, W2)
    q = _spmm_sc(s2, src_r, dst_r)
    h2 = _combine_final(q, b2r, h1)
    return (h1, h2)


# speedup vs baseline: 10.9851x; 3.3896x over previous
import functools

import jax
import jax.numpy as jnp
from jax import lax
from jax.experimental import pallas as pl
from jax.experimental.pallas import tpu as pltpu
from jax.experimental.pallas import tpu_sc as plsc

N = 10000
D = 128
NC = 2
NS = 16
NW = NC * NS
CHUNK = 128
IB = 8
N_ACC = 10240
ROWS_PER_TILE = N_ACC // NS
ZCH = ROWS_PER_TILE // CHUNK


def _spmm_sc(support, src_r, dst_r):
    C = src_r.shape[1]
    G = C // IB
    mesh = plsc.VectorSubcoreMesh(core_axis_name="c", subcore_axis_name="s")

    @functools.partial(
        pl.kernel,
        out_type=jax.ShapeDtypeStruct((NC, N_ACC, D), jnp.float32),
        mesh=mesh,
        scratch_types=[
            pltpu.VMEM((IB, CHUNK), jnp.int32),
            pltpu.VMEM((IB, CHUNK), jnp.int32),
            pltpu.VMEM((CHUNK, D), jnp.float32),
            pltpu.VMEM((CHUNK, D), jnp.float32),
            pltpu.VMEM_SHARED((N_ACC, D), jnp.float32),
            pltpu.SemaphoreType.DMA,
            pltpu.SemaphoreType.DMA,
        ],
    )
    def spmm(support_hbm, src_hbm, dst_hbm, out_hbm, sidx, didx, buf0, buf1,
             acc, sem0, sem1):
        c = lax.axis_index("c")
        s = lax.axis_index("s")
        wid = s * NC + c
        bufs = (buf0, buf1)
        sems = (sem0, sem1)

        def zrow(r, _):
            def zcol(q, _):
                buf0[r, pl.ds(q * 16, 16)] = jnp.zeros((16,), jnp.float32)
                return 0
            return lax.fori_loop(0, D // 16, zcol, 0)
        lax.fori_loop(0, CHUNK, zrow, 0)

        def zacc(i, _):
            off = s * ROWS_PER_TILE + i * CHUNK
            pltpu.sync_copy(buf0, acc.at[pl.ds(off, CHUNK)])
            return 0
        lax.fori_loop(0, ZCH, zacc, 0)
        plsc.subcore_barrier()

        def sup(g, _):
            pltpu.sync_copy(src_hbm.at[wid, pl.ds(g * IB, IB)], sidx)
            pltpu.sync_copy(dst_hbm.at[wid, pl.ds(g * IB, IB)], didx)
            descs = [
                pltpu.async_copy(support_hbm.at[sidx.at[0]], buf0, sem0),
                pltpu.async_copy(support_hbm.at[sidx.at[1]], buf1, sem1),
            ]
            for b in range(IB):
                k = b % 2
                descs[k].wait()
                pltpu.sync_copy(bufs[k], acc.at[didx.at[b]], add=True)
                if b + 2 < IB:
                    descs[k] = pltpu.async_copy(
                        support_hbm.at[sidx.at[b + 2]], bufs[k], sems[k])
            return 0
        lax.fori_loop(0, G, sup, 0)
        plsc.subcore_barrier()

        def cout(i, _):
            off = s * ROWS_PER_TILE + i * CHUNK
            pltpu.sync_copy(acc.at[pl.ds(off, CHUNK)], buf0)
            pltpu.sync_copy(buf0, out_hbm.at[c, pl.ds(off, CHUNK)])
            return 0
        lax.fori_loop(0, ZCH, cout, 0)

    return spmm(support, src_r, dst_r)


def _mm_tc(x, w):
    def body(x_ref, w_ref, o_ref):
        o_ref[...] = jnp.dot(x_ref[...], w_ref[...],
                             preferred_element_type=jnp.float32)
    return pl.pallas_call(
        body,
        out_shape=jax.ShapeDtypeStruct((x.shape[0], w.shape[1]), jnp.float32),
    )(x, w)


def _combine_bn_elu_mm(p, b, x, gamma, beta, w2):
    def body(p_ref, b_ref, x_ref, g_ref, be_ref, w2_ref, h1_ref, s2_ref):
        h = p_ref[0, :N, :] + p_ref[1, :N, :] + x_ref[...] + b_ref[...]
        mean = jnp.mean(h, axis=0, keepdims=True)
        var = jnp.mean((h - mean) * (h - mean), axis=0, keepdims=True)
        hn = g_ref[...] * (h - mean) * lax.rsqrt(var + 1e-5) + be_ref[...]
        h1 = jnp.where(hn > 0, hn, jnp.exp(jnp.minimum(hn, 0.0)) - 1.0)
        h1_ref[...] = h1
        s2_ref[...] = jnp.dot(h1, w2_ref[...],
                              preferred_element_type=jnp.float32)
    return pl.pallas_call(
        body,
        out_shape=(
            jax.ShapeDtypeStruct((N, D), jnp.float32),
            jax.ShapeDtypeStruct((N, D), jnp.float32),
        ),
    )(p, b, x, gamma, beta, w2)


def _combine_final(q, b, h1):
    def body(q_ref, b_ref, h1_ref, o_ref):
        o_ref[...] = q_ref[0, :N, :] + q_ref[1, :N, :] + b_ref[...] + h1_ref[...]
    return pl.pallas_call(
        body,
        out_shape=jax.ShapeDtypeStruct((N, D), jnp.float32),
    )(q, b, h1)


def kernel(features, edge_index, W1, b1, W2, b2, gamma0, beta0):
    E = edge_index.shape[1]
    C = -(-E // (NW * CHUNK * IB)) * IB
    E_pad = NW * C * CHUNK
    src = edge_index[0]
    dst = edge_index[1]
    pad = E_pad - E
    r = jnp.arange(pad, dtype=jnp.int32)
    src_p = jnp.concatenate([src, r % N])
    dst_p = jnp.concatenate([dst, N + (r % (N_ACC - N))])
    src_r = src_p.reshape(NW, C, CHUNK)
    dst_r = dst_p.reshape(NW, C, CHUNK)

    b1r = b1.reshape(1, D)
    b2r = b2.reshape(1, D)
    g0 = gamma0.reshape(1, D)
    be0 = beta0.reshape(1, D)

    s1 = _mm_tc(features, W1)
    p = _spmm_sc(s1, src_r, dst_r)
    h1, s2 = _combine_bn_elu_mm(p, b1r, features, g0, be0, W2)
    q = _spmm_sc(s2, src_r, dst_r)
    h2 = _combine_final(q, b2r, h1)
    return (h1, h2)


# speedup vs baseline: 13.5231x; 1.2310x over previous
import functools

import jax
import jax.numpy as jnp
from jax import lax
from jax.experimental import pallas as pl
from jax.experimental.pallas import tpu as pltpu
from jax.experimental.pallas import tpu_sc as plsc

N = 10000
D = 128
NC = 2
NS = 16
NW = NC * NS
CHUNK = 128
IB = 8
N_ACC = 10240
ROWS_PER_TILE = N_ACC // NS
ZCH = ROWS_PER_TILE // CHUNK


def _spmm_sc(support, src_r, dst_r):
    C = src_r.shape[1]
    G = C // IB
    assert C % (2 * IB) == 0
    mesh = plsc.VectorSubcoreMesh(core_axis_name="c", subcore_axis_name="s")

    @functools.partial(
        pl.kernel,
        out_type=jax.ShapeDtypeStruct((NC, N_ACC, D), jnp.float32),
        mesh=mesh,
        scratch_types=[
            pltpu.VMEM((2, IB, CHUNK), jnp.int32),
            pltpu.VMEM((2, IB, CHUNK), jnp.int32),
            pltpu.VMEM((CHUNK, D), jnp.float32),
            pltpu.VMEM((CHUNK, D), jnp.float32),
            pltpu.VMEM_SHARED((N_ACC, D), jnp.float32),
            pltpu.SemaphoreType.DMA,
            pltpu.SemaphoreType.DMA,
            pltpu.SemaphoreType.DMA,
            pltpu.SemaphoreType.DMA,
        ],
    )
    def spmm(support_hbm, src_hbm, dst_hbm, out_hbm, sidx, didx, buf0, buf1,
             acc, semr0, semr1, semi0, semi1):
        c = lax.axis_index("c")
        s = lax.axis_index("s")
        wid = s * NC + c
        bufs = (buf0, buf1)
        semr = (semr0, semr1)
        semi = (semi0, semi1)

        def fire_idx(block, slot):
            pltpu.async_copy(src_hbm.at[wid, pl.ds(block * IB, IB)],
                             sidx.at[slot], semi[slot])
            pltpu.async_copy(dst_hbm.at[wid, pl.ds(block * IB, IB)],
                             didx.at[slot], semi[slot])

        def wait_idx(slot):
            for _ in range(2):
                pltpu.make_async_copy(src_hbm.at[wid, pl.ds(0, IB)],
                                      sidx.at[slot], semi[slot]).wait()

        def wait_rows(k):
            pltpu.make_async_copy(support_hbm.at[pl.ds(0, CHUNK)],
                                  bufs[k], semr[k]).wait()

        fire_idx(0, 0)

        def zrow(r, _):
            def zcol(q, _):
                buf0[r, pl.ds(q * 16, 16)] = jnp.zeros((16,), jnp.float32)
                return 0
            return lax.fori_loop(0, D // 16, zcol, 0)
        lax.fori_loop(0, CHUNK, zrow, 0)

        def zacc(i, _):
            off = s * ROWS_PER_TILE + i * CHUNK
            pltpu.sync_copy(buf0, acc.at[pl.ds(off, CHUNK)])
            return 0
        lax.fori_loop(0, ZCH, zacc, 0)

        wait_idx(0)
        pltpu.async_copy(support_hbm.at[sidx.at[0, 0]], buf0, semr0)
        pltpu.async_copy(support_hbm.at[sidx.at[0, 1]], buf1, semr1)
        plsc.subcore_barrier()

        def sup(gg, _):
            for h in (0, 1):
                g = gg * 2 + h
                nxt = jnp.minimum(g + 1, G - 1)
                fire_idx(nxt, 1 - h)
                for b in range(IB):
                    k = b % 2
                    if b == IB - 2:
                        wait_idx(1 - h)
                    wait_rows(k)
                    pltpu.sync_copy(bufs[k], acc.at[didx.at[h, b]], add=True)
                    if b < IB - 2:
                        pltpu.async_copy(support_hbm.at[sidx.at[h, b + 2]],
                                         bufs[k], semr[k])
                    else:
                        @pl.when(g < G - 1)
                        def _():
                            pltpu.async_copy(
                                support_hbm.at[sidx.at[1 - h, b + 2 - IB]],
                                bufs[k], semr[k])
            return 0
        lax.fori_loop(0, G // 2, sup, 0)
        plsc.subcore_barrier()

        def rd(i, k):
            off = s * ROWS_PER_TILE + i * CHUNK
            return pltpu.async_copy(acc.at[pl.ds(off, CHUNK)], bufs[k],
                                    semr[k])
        rdesc = [rd(0, 0), rd(1, 1)]
        wdesc = [None, None]
        for i in range(ZCH):
            k = i % 2
            off = s * ROWS_PER_TILE + i * CHUNK
            rdesc[k].wait()
            wdesc[k] = pltpu.async_copy(
                bufs[k], out_hbm.at[c, pl.ds(off, CHUNK)], semi[k])
            if i + 2 < ZCH:
                wdesc[k].wait()
                rdesc[k] = rd(i + 2, k)
        wdesc[(ZCH - 2) % 2].wait()
        wdesc[(ZCH - 1) % 2].wait()

    return spmm(support, src_r, dst_r)


def _mm_tc(x, w):
    def body(x_ref, w_ref, o_ref):
        o_ref[...] = jnp.dot(x_ref[...], w_ref[...],
                             preferred_element_type=jnp.float32)
    return pl.pallas_call(
        body,
        out_shape=jax.ShapeDtypeStruct((x.shape[0], w.shape[1]), jnp.float32),
    )(x, w)


def _combine_bn_elu_mm(p, b, x, gamma, beta, w2):
    def body(p_ref, b_ref, x_ref, g_ref, be_ref, w2_ref, h1_ref, s2_ref):
        h = p_ref[0, :N, :] + p_ref[1, :N, :] + x_ref[...] + b_ref[...]
        mean = jnp.mean(h, axis=0, keepdims=True)
        var = jnp.mean((h - mean) * (h - mean), axis=0, keepdims=True)
        hn = g_ref[...] * (h - mean) * lax.rsqrt(var + 1e-5) + be_ref[...]
        h1 = jnp.where(hn > 0, hn, jnp.exp(jnp.minimum(hn, 0.0)) - 1.0)
        h1_ref[...] = h1
        s2_ref[...] = jnp.dot(h1, w2_ref[...],
                              preferred_element_type=jnp.float32)
    return pl.pallas_call(
        body,
        out_shape=(
            jax.ShapeDtypeStruct((N, D), jnp.float32),
            jax.ShapeDtypeStruct((N, D), jnp.float32),
        ),
    )(p, b, x, gamma, beta, w2)


def _combine_final(q, b, h1):
    def body(q_ref, b_ref, h1_ref, o_ref):
        o_ref[...] = q_ref[0, :N, :] + q_ref[1, :N, :] + b_ref[...] + h1_ref[...]
    return pl.pallas_call(
        body,
        out_shape=jax.ShapeDtypeStruct((N, D), jnp.float32),
    )(q, b, h1)


def kernel(features, edge_index, W1, b1, W2, b2, gamma0, beta0):
    E = edge_index.shape[1]
    ei = edge_index
    if E % CHUNK != 0:
        e_up = -(-E // CHUNK) * CHUNK
        ei = jnp.concatenate(
            [ei, jnp.stack([jnp.zeros((e_up - E,), jnp.int32),
                            jnp.full((e_up - E,), N, jnp.int32)])], axis=1)
        E = e_up
    T = E // CHUNK
    C = -(-T // (NW * 2 * IB)) * (2 * IB)
    pad_t = NW * C - T
    main = ei.reshape(2, T, CHUNK)
    r = jnp.arange(pad_t * CHUNK, dtype=jnp.int32).reshape(pad_t, CHUNK)
    pads = jnp.stack([r % N, N + (r % (N_ACC - N))])
    full = jnp.concatenate([main, pads], axis=1)
    src_r = full[0].reshape(NW, C, CHUNK)
    dst_r = full[1].reshape(NW, C, CHUNK)

    b1r = b1.reshape(1, D)
    b2r = b2.reshape(1, D)
    g0 = gamma0.reshape(1, D)
    be0 = beta0.reshape(1, D)

    s1 = _mm_tc(features, W1)
    p = _spmm_sc(s1, src_r, dst_r)
    h1, s2 = _combine_bn_elu_mm(p, b1r, features, g0, be0, W2)
    q = _spmm_sc(s2, src_r, dst_r)
    h2 = _combine_final(q, b2r, h1)
    return (h1, h2)
